# Initial kernel scaffold; baseline (speedup 1.0000x reference)
import jax
import jax.numpy as jnp
from jax.experimental import pallas as pl


def kernel(x, edge_index, W1, aS1, aD1, b1, W2, aS2, aD2, b2, W3, aS3, aD3, b3):
    raise NotImplementedError("write your pallas kernel here")


# speedup vs baseline: 58.3163x; 58.3163x over previous
import functools

import jax
import jax.numpy as jnp
from jax import lax
from jax.experimental import pallas as pl
from jax.experimental.pallas import tpu as pltpu
from jax.experimental.pallas import tpu_sc as plsc

NN = 10000
EE = 320000
KK = 80
BLK = 8
NBLOCKS = EE // (KK * BLK)
CH = 624
ZR = 156


def _lrelu(v):
    return jnp.where(v >= 0.0, v, 0.2 * v)


def _dense_first_body(x_ref, w_ref, asw_ref, adw_ref,
                      h_ref, asv_ref, adv_ref, ws_ref, g_ref):
    h = jnp.dot(x_ref[...], w_ref[...], preferred_element_type=jnp.float32)
    h_ref[...] = h
    asv = jnp.sum(h * asw_ref[...], axis=1, keepdims=True)
    adv = jnp.sum(h * adw_ref[...], axis=1, keepdims=True)
    asv_ref[...] = asv
    adv_ref[...] = adv
    g = _lrelu(jnp.max(asv) + jnp.max(adv))
    g_ref[...] = jnp.full((1, 1), g, jnp.float32)
    ws_ref[...] = jnp.exp(_lrelu(asv + adv) - g)


def _dense_first(x, w, asw, adw):
    c = w.shape[1]
    return pl.pallas_call(
        _dense_first_body,
        out_shape=[
            jax.ShapeDtypeStruct((NN, c), jnp.float32),
            jax.ShapeDtypeStruct((NN, 1), jnp.float32),
            jax.ShapeDtypeStruct((NN, 1), jnp.float32),
            jax.ShapeDtypeStruct((NN, 1), jnp.float32),
            jax.ShapeDtypeStruct((1, 1), jnp.float32),
        ],
    )(x, w, asw, adw)


def _dense_mid_body(acc_ref, den_ref, hp_ref, ws_ref, b_ref,
                    w_ref, asw_ref, adw_ref,
                    h_ref, asv_ref, adv_ref, wso_ref, g_ref):
    ws = ws_ref[...]
    acc = acc_ref[0] + acc_ref[1] + ws * hp_ref[...]
    den = den_ref[0] + den_ref[1] + ws
    o = acc / den + b_ref[...]
    xn = jnp.where(o > 0.0, o, jnp.exp(o) - 1.0)
    h = jnp.dot(xn, w_ref[...], preferred_element_type=jnp.float32)
    h_ref[...] = h
    asv = jnp.sum(h * asw_ref[...], axis=1, keepdims=True)
    adv = jnp.sum(h * adw_ref[...], axis=1, keepdims=True)
    asv_ref[...] = asv
    adv_ref[...] = adv
    g = _lrelu(jnp.max(asv) + jnp.max(adv))
    g_ref[...] = jnp.full((1, 1), g, jnp.float32)
    wso_ref[...] = jnp.exp(_lrelu(asv + adv) - g)


def _dense_mid(acc, den, hp, ws, b, w, asw, adw):
    c = w.shape[1]
    return pl.pallas_call(
        _dense_mid_body,
        out_shape=[
            jax.ShapeDtypeStruct((NN, c), jnp.float32),
            jax.ShapeDtypeStruct((NN, 1), jnp.float32),
            jax.ShapeDtypeStruct((NN, 1), jnp.float32),
            jax.ShapeDtypeStruct((NN, 1), jnp.float32),
            jax.ShapeDtypeStruct((1, 1), jnp.float32),
        ],
    )(acc, den, hp, ws, b, w, asw, adw)


def _epilogue_body(acc_ref, den_ref, hp_ref, ws_ref, b_ref, out_ref):
    ws = ws_ref[...]
    acc = acc_ref[0] + acc_ref[1] + ws * hp_ref[...]
    den = den_ref[0] + den_ref[1] + ws
    out_ref[...] = acc / den + b_ref[...]


def _epilogue(acc, den, hp, ws, b):
    c = hp.shape[1]
    return pl.pallas_call(
        _epilogue_body,
        out_shape=jax.ShapeDtypeStruct((NN, c), jnp.float32),
    )(acc, den, hp, ws, b)


def _make_edge_call(c_dim):
    mesh = plsc.VectorSubcoreMesh(core_axis_name="c", subcore_axis_name="s")

    @functools.partial(
        pl.kernel,
        mesh=mesh,
        out_type=[
            jax.ShapeDtypeStruct((2, NN, c_dim), jnp.float32),
            jax.ShapeDtypeStruct((2, NN), jnp.float32),
        ],
        scratch_types=[
            pltpu.VMEM_SHARED((NN, c_dim), jnp.float32),
            pltpu.VMEM_SHARED((NN,), jnp.float32),
            pltpu.VMEM_SHARED((NN,), jnp.float32),
            pltpu.VMEM_SHARED((NN, c_dim), jnp.float32),
            pltpu.VMEM_SHARED((NN,), jnp.float32),
            pltpu.VMEM((BLK, KK), jnp.int32),
            pltpu.VMEM((BLK, KK), jnp.int32),
            pltpu.VMEM((KK,), jnp.int32),
            pltpu.VMEM((KK,), jnp.int32),
            pltpu.VMEM((KK,), jnp.float32),
            pltpu.VMEM((KK,), jnp.float32),
            pltpu.VMEM((KK,), jnp.float32),
            pltpu.VMEM((KK, c_dim), jnp.float32),
            pltpu.VMEM((ZR, c_dim), jnp.float32),
            pltpu.VMEM((1000,), jnp.float32),
            pltpu.VMEM((16,), jnp.float32),
            pltpu.SemaphoreType.DMA,
            pltpu.SemaphoreType.DMA,
            pltpu.SemaphoreType.DMA,
        ],
    )
    def edge_call(h_hbm, as_hbm, ad_hbm, g_hbm, src_hbm, dst_hbm,
                  acc_out, den_out,
                  h_sp, as_sp, ad_sp, acc_sp, den_sp,
                  src_st, dst_st, sidx, didx, asv, adv, wbuf, rows,
                  zrows, z1d, gbuf, sem1, sem2, sem3):
        cid = lax.axis_index("c")
        sid = lax.axis_index("s")
        zv = jnp.zeros((16,), jnp.float32)

        pltpu.sync_copy(h_hbm.at[pl.ds(sid * CH, CH)],
                        h_sp.at[pl.ds(sid * CH, CH)])

        @pl.when(sid == 15)
        def _():
            pltpu.sync_copy(h_hbm.at[pl.ds(16 * CH, NN - 16 * CH)],
                            h_sp.at[pl.ds(16 * CH, NN - 16 * CH)])

        @pl.when(sid == 0)
        def _():
            pltpu.sync_copy(as_hbm, as_sp)
            pltpu.sync_copy(ad_hbm, ad_sp)

        def _zr(i, carry):
            for cc in range(c_dim // 16):
                zrows[i, pl.ds(cc * 16, 16)] = zv
            return carry
        lax.fori_loop(0, ZR, _zr, 0)

        def _z1(i, carry):
            z1d[pl.ds(i * 16, 16)] = zv
            return carry
        lax.fori_loop(0, 1000 // 16, _z1, 0)

        for t in range(4):
            pltpu.sync_copy(zrows, acc_sp.at[pl.ds(sid * CH + t * ZR, ZR)])

        @pl.when(sid == 15)
        def _():
            pltpu.sync_copy(zrows.at[pl.ds(0, NN - 16 * CH)],
                            acc_sp.at[pl.ds(16 * CH, NN - 16 * CH)])

        @pl.when(sid == 0)
        def _():
            for t in range(10):
                pltpu.sync_copy(z1d, den_sp.at[pl.ds(t * 1000, 1000)])

        pltpu.sync_copy(g_hbm, gbuf)
        plsc.subcore_barrier()

        gv = gbuf[...]
        wid = cid * 16 + sid
        start = jnp.where(wid < 20, 16 * wid, 320 + 15 * (wid - 20))
        cnt = jnp.where(wid < 20, 16, 15)

        def _blk(bi, carry):
            blk = start + bi
            pltpu.sync_copy(src_hbm.at[blk], src_st)
            pltpu.sync_copy(dst_hbm.at[blk], dst_st)

            for j in range(BLK):
                for t in range(KK // 16):
                    sidx[pl.ds(t * 16, 16)] = src_st[j, pl.ds(t * 16, 16)]
                    didx[pl.ds(t * 16, 16)] = dst_st[j, pl.ds(t * 16, 16)]
                cp_rows = pltpu.async_copy(h_sp.at[sidx], rows, sem1)
                cp_as = pltpu.async_copy(as_sp.at[sidx], asv, sem2)
                cp_ad = pltpu.async_copy(ad_sp.at[didx], adv, sem3)
                cp_as.wait()
                cp_ad.wait()
                for t in range(KK // 16):
                    e = asv[pl.ds(t * 16, 16)] + adv[pl.ds(t * 16, 16)]
                    wbuf[pl.ds(t * 16, 16)] = jnp.exp(_lrelu(e) - gv)
                cp_rows.wait()

                for t in range(KK // 16):
                    wv = wbuf[pl.ds(t * 16, 16)]
                    for ll in range(16):
                        p = t * 16 + ll
                        w_s = wv[ll]
                        for cc in range(c_dim // 16):
                            rows[p, pl.ds(cc * 16, 16)] = (
                                rows[p, pl.ds(cc * 16, 16)] * w_s)

                pltpu.sync_copy(rows, acc_sp.at[didx], add=True)
                pltpu.sync_copy(wbuf, den_sp.at[didx], add=True)
            return carry
        lax.fori_loop(0, cnt, _blk, 0)

        plsc.subcore_barrier()

        pltpu.sync_copy(acc_sp.at[pl.ds(sid * CH, CH)],
                        acc_out.at[cid, pl.ds(sid * CH, CH)])

        @pl.when(sid == 15)
        def _():
            pltpu.sync_copy(acc_sp.at[pl.ds(16 * CH, NN - 16 * CH)],
                            acc_out.at[cid, pl.ds(16 * CH, NN - 16 * CH)])

        @pl.when(sid == 0)
        def _():
            pltpu.sync_copy(den_sp, den_out.at[cid])

    return edge_call


_edge_call_16 = _make_edge_call(16)
_edge_call_64 = _make_edge_call(64)


def kernel(x, edge_index, W1, aS1, aD1, b1, W2, aS2, aD2, b2,
           W3, aS3, aD3, b3):
    src = edge_index[0].astype(jnp.int32).reshape(NBLOCKS, BLK, KK)
    dst = edge_index[1].astype(jnp.int32).reshape(NBLOCKS, BLK, KK)

    def g16(g):
        return jnp.broadcast_to(g.reshape(()), (16,))

    h1, as1, ad1, ws1, g1 = _dense_first(
        x, W1, aS1.reshape(1, -1), aD1.reshape(1, -1))
    acc1, den1 = _edge_call_16(h1, as1.reshape(NN), ad1.reshape(NN),
                               g16(g1), src, dst)

    h2, as2, ad2, ws2, g2 = _dense_mid(
        acc1, den1.reshape(2, NN, 1), h1, ws1, b1.reshape(1, -1),
        W2, aS2.reshape(1, -1), aD2.reshape(1, -1))
    acc2, den2 = _edge_call_16(h2, as2.reshape(NN), ad2.reshape(NN),
                               g16(g2), src, dst)

    h3, as3, ad3, ws3, g3 = _dense_mid(
        acc2, den2.reshape(2, NN, 1), h2, ws2, b2.reshape(1, -1),
        W3, aS3.reshape(1, -1), aD3.reshape(1, -1))
    acc3, den3 = _edge_call_64(h3, as3.reshape(NN), ad3.reshape(NN),
                               g16(g3), src, dst)

    return _epilogue(acc3, den3.reshape(2, NN, 1), h3, ws3,
                     b3.reshape(1, -1))


# speedup vs baseline: 65.0876x; 1.1161x over previous
import functools

import jax
import jax.numpy as jnp
from jax import lax
from jax.experimental import pallas as pl
from jax.experimental.pallas import tpu as pltpu
from jax.experimental.pallas import tpu_sc as plsc

NN = 10000
EE = 320000
KK = 80
BLK = 8
NBLOCKS = EE // (KK * BLK)
CH = 624
ZR = 156


def _lrelu(v):
    return jnp.where(v >= 0.0, v, 0.2 * v)


def _dense_first_body(x_ref, w_ref, asw_ref, adw_ref,
                      h_ref, asv_ref, adv_ref, ws_ref, g_ref):
    h = jnp.dot(x_ref[...], w_ref[...], preferred_element_type=jnp.float32)
    h_ref[...] = h
    asv = jnp.sum(h * asw_ref[...], axis=1, keepdims=True)
    adv = jnp.sum(h * adw_ref[...], axis=1, keepdims=True)
    asv_ref[...] = asv
    adv_ref[...] = adv
    g = _lrelu(jnp.max(asv) + jnp.max(adv))
    g_ref[...] = jnp.full((1, 1), g, jnp.float32)
    ws_ref[...] = jnp.exp(_lrelu(asv + adv) - g)


def _dense_first(x, w, asw, adw):
    c = w.shape[1]
    return pl.pallas_call(
        _dense_first_body,
        out_shape=[
            jax.ShapeDtypeStruct((NN, c), jnp.float32),
            jax.ShapeDtypeStruct((NN, 1), jnp.float32),
            jax.ShapeDtypeStruct((NN, 1), jnp.float32),
            jax.ShapeDtypeStruct((NN, 1), jnp.float32),
            jax.ShapeDtypeStruct((1, 1), jnp.float32),
        ],
    )(x, w, asw, adw)


def _dense_mid_body(acc_ref, den_ref, hp_ref, ws_ref, b_ref,
                    w_ref, asw_ref, adw_ref,
                    h_ref, asv_ref, adv_ref, wso_ref, g_ref):
    ws = ws_ref[...]
    acc = acc_ref[0] + acc_ref[1] + ws * hp_ref[...]
    den = den_ref[0] + den_ref[1] + ws
    o = acc / den + b_ref[...]
    xn = jnp.where(o > 0.0, o, jnp.exp(o) - 1.0)
    h = jnp.dot(xn, w_ref[...], preferred_element_type=jnp.float32)
    h_ref[...] = h
    asv = jnp.sum(h * asw_ref[...], axis=1, keepdims=True)
    adv = jnp.sum(h * adw_ref[...], axis=1, keepdims=True)
    asv_ref[...] = asv
    adv_ref[...] = adv
    g = _lrelu(jnp.max(asv) + jnp.max(adv))
    g_ref[...] = jnp.full((1, 1), g, jnp.float32)
    wso_ref[...] = jnp.exp(_lrelu(asv + adv) - g)


def _dense_mid(acc, den, hp, ws, b, w, asw, adw):
    c = w.shape[1]
    return pl.pallas_call(
        _dense_mid_body,
        out_shape=[
            jax.ShapeDtypeStruct((NN, c), jnp.float32),
            jax.ShapeDtypeStruct((NN, 1), jnp.float32),
            jax.ShapeDtypeStruct((NN, 1), jnp.float32),
            jax.ShapeDtypeStruct((NN, 1), jnp.float32),
            jax.ShapeDtypeStruct((1, 1), jnp.float32),
        ],
    )(acc, den, hp, ws, b, w, asw, adw)


def _epilogue_body(acc_ref, den_ref, hp_ref, ws_ref, b_ref, out_ref):
    ws = ws_ref[...]
    acc = acc_ref[0] + acc_ref[1] + ws * hp_ref[...]
    den = den_ref[0] + den_ref[1] + ws
    out_ref[...] = acc / den + b_ref[...]


def _epilogue(acc, den, hp, ws, b):
    c = hp.shape[1]
    return pl.pallas_call(
        _epilogue_body,
        out_shape=jax.ShapeDtypeStruct((NN, c), jnp.float32),
    )(acc, den, hp, ws, b)


def _make_edge_call(c_dim):
    mesh = plsc.VectorSubcoreMesh(core_axis_name="c", subcore_axis_name="s")

    @functools.partial(
        pl.kernel,
        mesh=mesh,
        out_type=[
            jax.ShapeDtypeStruct((2, NN, c_dim), jnp.float32),
            jax.ShapeDtypeStruct((2, NN), jnp.float32),
        ],
        scratch_types=[
            pltpu.VMEM_SHARED((NN, c_dim), jnp.float32),
            pltpu.VMEM_SHARED((NN,), jnp.float32),
            pltpu.VMEM_SHARED((NN,), jnp.float32),
            pltpu.VMEM_SHARED((NN, c_dim), jnp.float32),
            pltpu.VMEM_SHARED((NN,), jnp.float32),
            pltpu.VMEM((BLK, KK), jnp.int32),
            pltpu.VMEM((BLK, KK), jnp.int32),
            pltpu.VMEM((KK,), jnp.int32),
            pltpu.VMEM((KK,), jnp.int32),
            pltpu.VMEM((KK,), jnp.int32),
            pltpu.VMEM((KK,), jnp.int32),
            pltpu.VMEM((KK,), jnp.float32),
            pltpu.VMEM((KK,), jnp.float32),
            pltpu.VMEM((KK,), jnp.float32),
            pltpu.VMEM((KK,), jnp.float32),
            pltpu.VMEM((KK,), jnp.float32),
            pltpu.VMEM((KK,), jnp.float32),
            pltpu.VMEM((KK, c_dim), jnp.float32),
            pltpu.VMEM((KK, c_dim), jnp.float32),
            pltpu.VMEM((ZR, c_dim), jnp.float32),
            pltpu.VMEM((1000,), jnp.float32),
            pltpu.VMEM((16,), jnp.float32),
            pltpu.SemaphoreType.DMA,
            pltpu.SemaphoreType.DMA,
            pltpu.SemaphoreType.DMA,
            pltpu.SemaphoreType.DMA,
            pltpu.SemaphoreType.DMA,
            pltpu.SemaphoreType.DMA,
            pltpu.SemaphoreType.DMA,
            pltpu.SemaphoreType.DMA,
        ],
    )
    def edge_call(h_hbm, as_hbm, ad_hbm, g_hbm, src_hbm, dst_hbm,
                  acc_out, den_out,
                  h_sp, as_sp, ad_sp, acc_sp, den_sp,
                  src_st, dst_st, sidx0, sidx1, didx0, didx1,
                  asv0, asv1, adv0, adv1, wbuf0, wbuf1, rows0, rows1,
                  zrows, z1d, gbuf,
                  sem_r0, sem_r1, sem_a0, sem_a1, sem_d0, sem_d1,
                  sem_s0, sem_s1):
        sidx = [sidx0, sidx1]
        didx = [didx0, didx1]
        asv = [asv0, asv1]
        adv = [adv0, adv1]
        wbuf = [wbuf0, wbuf1]
        rows = [rows0, rows1]
        sem_r = [sem_r0, sem_r1]
        sem_a = [sem_a0, sem_a1]
        sem_d = [sem_d0, sem_d1]
        sem_s = [sem_s0, sem_s1]
        cid = lax.axis_index("c")
        sid = lax.axis_index("s")
        zv = jnp.zeros((16,), jnp.float32)

        pltpu.sync_copy(h_hbm.at[pl.ds(sid * CH, CH)],
                        h_sp.at[pl.ds(sid * CH, CH)])

        @pl.when(sid == 15)
        def _():
            pltpu.sync_copy(h_hbm.at[pl.ds(16 * CH, NN - 16 * CH)],
                            h_sp.at[pl.ds(16 * CH, NN - 16 * CH)])

        @pl.when(sid == 0)
        def _():
            pltpu.sync_copy(as_hbm, as_sp)
            pltpu.sync_copy(ad_hbm, ad_sp)

        def _zr(i, carry):
            for cc in range(c_dim // 16):
                zrows[i, pl.ds(cc * 16, 16)] = zv
            return carry
        lax.fori_loop(0, ZR, _zr, 0)

        def _z1(i, carry):
            z1d[pl.ds(i * 16, 16)] = zv
            return carry
        lax.fori_loop(0, 1000 // 16, _z1, 0)

        for t in range(4):
            pltpu.sync_copy(zrows, acc_sp.at[pl.ds(sid * CH + t * ZR, ZR)])

        @pl.when(sid == 15)
        def _():
            pltpu.sync_copy(zrows.at[pl.ds(0, NN - 16 * CH)],
                            acc_sp.at[pl.ds(16 * CH, NN - 16 * CH)])

        @pl.when(sid == 0)
        def _():
            for t in range(10):
                pltpu.sync_copy(z1d, den_sp.at[pl.ds(t * 1000, 1000)])

        pltpu.sync_copy(g_hbm, gbuf)
        plsc.subcore_barrier()

        gv = gbuf[...]
        wid = cid * 16 + sid
        start = jnp.where(wid < 20, 16 * wid, 320 + 15 * (wid - 20))
        cnt = jnp.where(wid < 20, 16, 15)

        def _prep(j, b):
            for t in range(KK // 16):
                sidx[b][pl.ds(t * 16, 16)] = src_st[j, pl.ds(t * 16, 16)]
                didx[b][pl.ds(t * 16, 16)] = dst_st[j, pl.ds(t * 16, 16)]
            return (
                pltpu.async_copy(h_sp.at[sidx[b]], rows[b], sem_r[b]),
                pltpu.async_copy(as_sp.at[sidx[b]], asv[b], sem_a[b]),
                pltpu.async_copy(ad_sp.at[didx[b]], adv[b], sem_d[b]),
            )

        def _blk(bi, carry):
            blk = start + bi
            pltpu.sync_copy(src_hbm.at[blk], src_st)
            pltpu.sync_copy(dst_hbm.at[blk], dst_st)

            gath = [None, None]
            gath[0] = _prep(0, 0)
            for j in range(BLK):
                b = j & 1
                nb = 1 - b
                if j + 1 < BLK:
                    gath[nb] = _prep(j + 1, nb)
                cp_rows, cp_as, cp_ad = gath[b]
                cp_as.wait()
                cp_ad.wait()
                for t in range(KK // 16):
                    e = asv[b][pl.ds(t * 16, 16)] + adv[b][pl.ds(t * 16, 16)]
                    wbuf[b][pl.ds(t * 16, 16)] = jnp.exp(_lrelu(e) - gv)
                cp_rows.wait()

                for t in range(KK // 16):
                    wv = wbuf[b][pl.ds(t * 16, 16)]
                    for ll in range(16):
                        p = t * 16 + ll
                        w_s = wv[ll]
                        for cc in range(c_dim // 16):
                            rows[b][p, pl.ds(cc * 16, 16)] = (
                                rows[b][p, pl.ds(cc * 16, 16)] * w_s)

                pltpu.sync_copy(rows[b], acc_sp.at[didx[b]], add=True)
                pltpu.sync_copy(wbuf[b], den_sp.at[didx[b]], add=True)
            return carry
        lax.fori_loop(0, cnt, _blk, 0)

        plsc.subcore_barrier()

        pltpu.sync_copy(acc_sp.at[pl.ds(sid * CH, CH)],
                        acc_out.at[cid, pl.ds(sid * CH, CH)])

        @pl.when(sid == 15)
        def _():
            pltpu.sync_copy(acc_sp.at[pl.ds(16 * CH, NN - 16 * CH)],
                            acc_out.at[cid, pl.ds(16 * CH, NN - 16 * CH)])

        @pl.when(sid == 0)
        def _():
            pltpu.sync_copy(den_sp, den_out.at[cid])

    return edge_call


_edge_call_16 = _make_edge_call(16)
_edge_call_64 = _make_edge_call(64)


def kernel(x, edge_index, W1, aS1, aD1, b1, W2, aS2, aD2, b2,
           W3, aS3, aD3, b3):
    src = edge_index[0].astype(jnp.int32).reshape(NBLOCKS, BLK, KK)
    dst = edge_index[1].astype(jnp.int32).reshape(NBLOCKS, BLK, KK)

    def g16(g):
        return jnp.broadcast_to(g.reshape(()), (16,))

    h1, as1, ad1, ws1, g1 = _dense_first(
        x, W1, aS1.reshape(1, -1), aD1.reshape(1, -1))
    acc1, den1 = _edge_call_16(h1, as1.reshape(NN), ad1.reshape(NN),
                               g16(g1), src, dst)

    h2, as2, ad2, ws2, g2 = _dense_mid(
        acc1, den1.reshape(2, NN, 1), h1, ws1, b1.reshape(1, -1),
        W2, aS2.reshape(1, -1), aD2.reshape(1, -1))
    acc2, den2 = _edge_call_16(h2, as2.reshape(NN), ad2.reshape(NN),
                               g16(g2), src, dst)

    h3, as3, ad3, ws3, g3 = _dense_mid(
        acc2, den2.reshape(2, NN, 1), h2, ws2, b2.reshape(1, -1),
        W3, aS3.reshape(1, -1), aD3.reshape(1, -1))
    acc3, den3 = _edge_call_64(h3, as3.reshape(NN), ad3.reshape(NN),
                               g16(g3), src, dst)

    return _epilogue(acc3, den3.reshape(2, NN, 1), h3, ws3,
                     b3.reshape(1, -1))
